# baseline (speedup 1.0000x reference)
import functools

import jax
import jax.numpy as jnp
from jax import lax
from jax.experimental import pallas as pl
from jax.experimental.pallas import tpu as pltpu
from jax.experimental.pallas import tpu_sc as plsc

N, E, D, L, B = 10000, 320000, 128, 3, 400
NC, NS = 2, 16
NW = NC * NS
C = 120
CA = 144
CB = 24
NBUF = 3
CHT = CA + CB
EP = NS * CHT * C
HP = 10016
NEG = -1.0e9


def _sc_edge_body(idx_hbm, g_hbm, zeros_hbm, out_hbm,
                  idx_r, rows0, rows1, rows2,
                  agg_sh, sem0, sem1, sem2):
    rows = (rows0, rows1, rows2)
    sems = (sem0, sem1, sem2)
    cid = lax.axis_index("c")
    sid = lax.axis_index("s")
    off = sid * CHT + cid * CA
    n_groups = jnp.where(cid == 0, CA // NBUF - 1, CB // NBUF - 1)

    @pl.when(sid == 0)
    def _init():
        pltpu.sync_copy(zeros_hbm, agg_sh)

    plsc.subcore_barrier()

    for b in range(NBUF):
        pltpu.sync_copy(idx_hbm.at[off + b], idx_r.at[b])
        pltpu.async_copy(g_hbm.at[idx_r.at[b, 0]], rows[b], sems[b])

    def group(gi, carry):
        for b in range(NBUF):
            j = gi * NBUF + b
            pltpu.make_async_copy(g_hbm.at[idx_r.at[b, 0]], rows[b], sems[b]).wait()
            pltpu.sync_copy(rows[b], agg_sh.at[idx_r.at[b, 1]], add=True)
            pltpu.sync_copy(idx_hbm.at[off + j + NBUF], idx_r.at[b])
            pltpu.async_copy(g_hbm.at[idx_r.at[b, 0]], rows[b], sems[b])
        return carry

    lax.fori_loop(0, n_groups, group, 0)
    for b in range(NBUF):
        pltpu.make_async_copy(g_hbm.at[idx_r.at[b, 0]], rows[b], sems[b]).wait()
        pltpu.sync_copy(rows[b], agg_sh.at[idx_r.at[b, 1]], add=True)

    plsc.subcore_barrier()

    @pl.when(sid == 0)
    def _writeback():
        pltpu.sync_copy(agg_sh, out_hbm.at[cid])


_sc_edge = pl.kernel(
    _sc_edge_body,
    out_type=jax.ShapeDtypeStruct((NC, HP, D), jnp.float32),
    mesh=plsc.VectorSubcoreMesh(core_axis_name="c", subcore_axis_name="s"),
    scratch_types=(
        [pltpu.VMEM((NBUF, 2, C), jnp.int32)]
        + [pltpu.VMEM((C, D), jnp.float32) for _ in range(NBUF)]
        + [pltpu.VMEM_SHARED((HP, D), jnp.float32)]
        + [pltpu.SemaphoreType.DMA for _ in range(NBUF)]
    ),
)


def _embed_body(x0_ref, ae_ref, out_ref):
    ids = lax.broadcasted_iota(jnp.int32, (1, 128), 1)
    oh = (x0_ref[...] == ids).astype(jnp.float32)
    h = jnp.dot(oh, ae_ref[...], preferred_element_type=jnp.float32,
                precision=lax.Precision.HIGHEST)
    rid = lax.broadcasted_iota(jnp.int32, (HP, 1), 0)
    out_ref[...] = jnp.where(rid < N, h, NEG)


_embed = pl.pallas_call(
    _embed_body,
    out_shape=jax.ShapeDtypeStruct((HP, D), jnp.float32),
)


def _makeg_body(h_ref, bond_ref, out_ref):
    b = pl.program_id(0)
    row = bond_ref[pl.ds(b, 1), :]
    out_ref[...] = jnp.maximum(h_ref[...] + row, 0.0)


_makeg = pl.pallas_call(
    _makeg_body,
    grid=(8,),
    in_specs=[
        pl.BlockSpec((HP, D), lambda b: (0, 0)),
        pl.BlockSpec((8, D), lambda b: (0, 0)),
    ],
    out_specs=pl.BlockSpec((HP, D), lambda b: (b, 0)),
    out_shape=jax.ShapeDtypeStruct((8 * HP, D), jnp.float32),
)


def _mlp_bn(h_ref, agg_ref, w1_ref, b1_ref, w2_ref, b2_ref,
            eps_ref, gam_ref, bet_ref):
    h = h_ref[:N, :]
    z = (1.0 + eps_ref[0, 0]) * h + agg_ref[0, :N, :] + agg_ref[1, :N, :]
    z = jnp.dot(z, w1_ref[...], preferred_element_type=jnp.float32) + b1_ref[...]
    z = jnp.maximum(z, 0.0)
    z = jnp.dot(z, w2_ref[...], preferred_element_type=jnp.float32) + b2_ref[...]
    mu = jnp.mean(z, axis=0, keepdims=True)
    zc = z - mu
    var = jnp.mean(zc * zc, axis=0, keepdims=True)
    return gam_ref[...] * zc * lax.rsqrt(var + 1e-5) + bet_ref[...]


def _dense_mid_body(h_ref, agg_ref, w1_ref, b1_ref, w2_ref, b2_ref,
                    eps_ref, gam_ref, bet_ref, out_ref):
    zn = _mlp_bn(h_ref, agg_ref, w1_ref, b1_ref, w2_ref, b2_ref,
                 eps_ref, gam_ref, bet_ref)
    out_ref[:N, :] = jnp.maximum(zn, 0.0)
    out_ref[N:, :] = jnp.full((HP - N, D), NEG, jnp.float32)


_dense_mid = pl.pallas_call(
    _dense_mid_body,
    out_shape=jax.ShapeDtypeStruct((HP, D), jnp.float32),
)


def _dense_pool_body(h_ref, agg_ref, w1_ref, b1_ref, w2_ref, b2_ref,
                     eps_ref, gam_ref, bet_ref, b_ref, out_ref):
    zn = _mlp_bn(h_ref, agg_ref, w1_ref, b1_ref, w2_ref, b2_ref,
                 eps_ref, gam_ref, bet_ref)
    ids = lax.broadcasted_iota(jnp.int32, (1, B), 1)
    P = (b_ref[:N, :] == ids).astype(jnp.float32)
    sums = lax.dot_general(P, zn, (((0,), (0,)), ((), ())),
                           preferred_element_type=jnp.float32,
                           precision=lax.Precision.HIGHEST)
    counts = jnp.sum(P, axis=0)[:, None]
    out_ref[...] = sums / jnp.maximum(counts, 1.0)


_dense_pool = pl.pallas_call(
    _dense_pool_body,
    out_shape=jax.ShapeDtypeStruct((B, D), jnp.float32),
)


def kernel(x, edge_index, edge_attr, batch, atom_embed, bond_embed,
           W1, b1, W2, b2, eps, gamma, beta):
    f32 = jnp.float32
    x0 = jnp.pad(x[:, 0].astype(jnp.int32)[:, None], ((0, HP - N), (0, 0)),
                 constant_values=127)
    src = edge_index[0].astype(jnp.int32)
    dst = edge_index[1].astype(jnp.int32)
    attr = edge_attr[:, 0].astype(jnp.int32)
    cidx = jnp.pad(attr * HP + src, (0, EP - E),
                   constant_values=N).reshape(NS * CHT, C)
    pad_dst = N + (jnp.arange(EP - E, dtype=jnp.int32) % (HP - N))
    dstp = jnp.concatenate([dst, pad_dst]).reshape(NS * CHT, C)
    idx2 = jnp.stack([cidx, dstp], axis=1)
    ae_pad = jnp.pad(atom_embed.astype(f32), ((0, 128 - 119), (0, 0)))
    bond = bond_embed.astype(f32)
    zeros = jnp.zeros((HP, D), f32)
    batch_p = jnp.pad(batch.astype(jnp.int32)[:, None], ((0, HP - N), (0, 0)),
                      constant_values=B)

    h = _embed(x0, ae_pad)
    out = None
    for l in range(L):
        g = _makeg(h, bond)
        agg2 = _sc_edge(idx2, g, zeros)
        args = (h, agg2, W1[l], b1[l][None, :], W2[l], b2[l][None, :],
                eps[l].reshape(1, 1), gamma[l][None, :], beta[l][None, :])
        if l == L - 1:
            out = _dense_pool(*args, batch_p)
        else:
            h = _dense_mid(*args)
    return out


# speedup vs baseline: 1.0261x; 1.0261x over previous
import functools

import jax
import jax.numpy as jnp
from jax import lax
from jax.experimental import pallas as pl
from jax.experimental.pallas import tpu as pltpu
from jax.experimental.pallas import tpu_sc as plsc

N, E, D, L, B = 10000, 320000, 128, 3, 400
NC, NS = 2, 16
NW = NC * NS
C = 120
CA = 138
CB = 30
NBUF = 3
CHT = CA + CB
EP = NS * CHT * C
HP = 10016
NEG = -1.0e9


def _sc_edge_body(idx_hbm, g_hbm, zeros_hbm, out_hbm,
                  idx_r, rows0, rows1, rows2,
                  agg_sh, sem0, sem1, sem2):
    rows = (rows0, rows1, rows2)
    sems = (sem0, sem1, sem2)
    cid = lax.axis_index("c")
    sid = lax.axis_index("s")
    off = sid * CHT + cid * CA
    n_groups = jnp.where(cid == 0, CA // NBUF - 1, CB // NBUF - 1)

    @pl.when(sid == 0)
    def _init():
        pltpu.sync_copy(zeros_hbm, agg_sh)

    plsc.subcore_barrier()

    for b in range(NBUF):
        pltpu.sync_copy(idx_hbm.at[off + b], idx_r.at[b])
        pltpu.async_copy(g_hbm.at[idx_r.at[b, 0]], rows[b], sems[b])

    def group(gi, carry):
        for b in range(NBUF):
            j = gi * NBUF + b
            pltpu.make_async_copy(g_hbm.at[idx_r.at[b, 0]], rows[b], sems[b]).wait()
            pltpu.sync_copy(rows[b], agg_sh.at[idx_r.at[b, 1]], add=True)
            pltpu.sync_copy(idx_hbm.at[off + j + NBUF], idx_r.at[b])
            pltpu.async_copy(g_hbm.at[idx_r.at[b, 0]], rows[b], sems[b])
        return carry

    lax.fori_loop(0, n_groups, group, 0)
    for b in range(NBUF):
        pltpu.make_async_copy(g_hbm.at[idx_r.at[b, 0]], rows[b], sems[b]).wait()
        pltpu.sync_copy(rows[b], agg_sh.at[idx_r.at[b, 1]], add=True)

    plsc.subcore_barrier()

    @pl.when(sid == 0)
    def _writeback():
        pltpu.sync_copy(agg_sh, out_hbm.at[cid])


_sc_edge = pl.kernel(
    _sc_edge_body,
    out_type=jax.ShapeDtypeStruct((NC, HP, D), jnp.float32),
    mesh=plsc.VectorSubcoreMesh(core_axis_name="c", subcore_axis_name="s"),
    scratch_types=(
        [pltpu.VMEM((NBUF, 2, C), jnp.int32)]
        + [pltpu.VMEM((C, D), jnp.float32) for _ in range(NBUF)]
        + [pltpu.VMEM_SHARED((HP, D), jnp.float32)]
        + [pltpu.SemaphoreType.DMA for _ in range(NBUF)]
    ),
)


def _embed_body(x0_ref, ae_ref, out_ref):
    ids = lax.broadcasted_iota(jnp.int32, (1, 128), 1)
    oh = (x0_ref[...] == ids).astype(jnp.float32)
    h = jnp.dot(oh, ae_ref[...], preferred_element_type=jnp.float32,
                precision=lax.Precision.HIGHEST)
    rid = lax.broadcasted_iota(jnp.int32, (HP, 1), 0)
    out_ref[...] = jnp.where(rid < N, h, NEG)


_embed = pl.pallas_call(
    _embed_body,
    out_shape=jax.ShapeDtypeStruct((HP, D), jnp.float32),
)


def _makeg_body(h_ref, bond_ref, out_ref):
    b = pl.program_id(0)
    row = bond_ref[pl.ds(b, 1), :]
    out_ref[...] = jnp.maximum(h_ref[...] + row, 0.0)


_makeg = pl.pallas_call(
    _makeg_body,
    grid=(8,),
    in_specs=[
        pl.BlockSpec((HP, D), lambda b: (0, 0)),
        pl.BlockSpec((8, D), lambda b: (0, 0)),
    ],
    out_specs=pl.BlockSpec((HP, D), lambda b: (b, 0)),
    out_shape=jax.ShapeDtypeStruct((8 * HP, D), jnp.float32),
)


def _mlp_bn(h_ref, agg_ref, w1_ref, b1_ref, w2_ref, b2_ref,
            eps_ref, gam_ref, bet_ref):
    h = h_ref[:N, :]
    z = (1.0 + eps_ref[0, 0]) * h + agg_ref[0, :N, :] + agg_ref[1, :N, :]
    z = jnp.dot(z, w1_ref[...], preferred_element_type=jnp.float32) + b1_ref[...]
    z = jnp.maximum(z, 0.0)
    z = jnp.dot(z, w2_ref[...], preferred_element_type=jnp.float32) + b2_ref[...]
    mu = jnp.mean(z, axis=0, keepdims=True)
    zc = z - mu
    var = jnp.mean(zc * zc, axis=0, keepdims=True)
    return gam_ref[...] * zc * lax.rsqrt(var + 1e-5) + bet_ref[...]


def _dense_mid_body(h_ref, agg_ref, w1_ref, b1_ref, w2_ref, b2_ref,
                    eps_ref, gam_ref, bet_ref, out_ref):
    zn = _mlp_bn(h_ref, agg_ref, w1_ref, b1_ref, w2_ref, b2_ref,
                 eps_ref, gam_ref, bet_ref)
    out_ref[:N, :] = jnp.maximum(zn, 0.0)
    out_ref[N:, :] = jnp.full((HP - N, D), NEG, jnp.float32)


_dense_mid = pl.pallas_call(
    _dense_mid_body,
    out_shape=jax.ShapeDtypeStruct((HP, D), jnp.float32),
)


def _dense_pool_body(h_ref, agg_ref, w1_ref, b1_ref, w2_ref, b2_ref,
                     eps_ref, gam_ref, bet_ref, b_ref, out_ref):
    zn = _mlp_bn(h_ref, agg_ref, w1_ref, b1_ref, w2_ref, b2_ref,
                 eps_ref, gam_ref, bet_ref)
    ids = lax.broadcasted_iota(jnp.int32, (1, B), 1)
    P = (b_ref[:N, :] == ids).astype(jnp.float32)
    sums = lax.dot_general(P, zn, (((0,), (0,)), ((), ())),
                           preferred_element_type=jnp.float32,
                           precision=lax.Precision.HIGHEST)
    counts = jnp.sum(P, axis=0)[:, None]
    out_ref[...] = sums / jnp.maximum(counts, 1.0)


_dense_pool = pl.pallas_call(
    _dense_pool_body,
    out_shape=jax.ShapeDtypeStruct((B, D), jnp.float32),
)


def kernel(x, edge_index, edge_attr, batch, atom_embed, bond_embed,
           W1, b1, W2, b2, eps, gamma, beta):
    f32 = jnp.float32
    x0 = jnp.pad(x[:, 0].astype(jnp.int32)[:, None], ((0, HP - N), (0, 0)),
                 constant_values=127)
    src = edge_index[0].astype(jnp.int32)
    dst = edge_index[1].astype(jnp.int32)
    attr = edge_attr[:, 0].astype(jnp.int32)
    cidx = jnp.pad(attr * HP + src, (0, EP - E),
                   constant_values=N).reshape(NS * CHT, C)
    pad_dst = N + (jnp.arange(EP - E, dtype=jnp.int32) % (HP - N))
    dstp = jnp.concatenate([dst, pad_dst]).reshape(NS * CHT, C)
    idx2 = jnp.stack([cidx, dstp], axis=1)
    ae_pad = jnp.pad(atom_embed.astype(f32), ((0, 128 - 119), (0, 0)))
    bond = bond_embed.astype(f32)
    zeros = jnp.zeros((HP, D), f32)
    batch_p = jnp.pad(batch.astype(jnp.int32)[:, None], ((0, HP - N), (0, 0)),
                      constant_values=B)

    h = _embed(x0, ae_pad)
    out = None
    for l in range(L):
        g = _makeg(h, bond)
        agg2 = _sc_edge(idx2, g, zeros)
        args = (h, agg2, W1[l], b1[l][None, :], W2[l], b2[l][None, :],
                eps[l].reshape(1, 1), gamma[l][None, :], beta[l][None, :])
        if l == L - 1:
            out = _dense_pool(*args, batch_p)
        else:
            h = _dense_mid(*args)
    return out
